# baseline (speedup 1.0000x reference)
import functools

import jax
import jax.numpy as jnp
from jax.experimental import pallas as pl
from jax.experimental.pallas import tpu as pltpu

_EPS = 1e-5
_VMEM_LIMIT = 32 * 1024 * 1024
_IMGS_PER_STEP = 8


def _sigmoid(t):
    return 1.0 / (1.0 + jnp.exp(-t))


def _silu(t):
    return t * _sigmoid(t)


def _fold_bn_from_partials(st, count, gamma, beta):
    tot = jnp.sum(st, axis=0)
    mean = tot[0:1] / count
    var = jnp.maximum(tot[1:2] / count - mean * mean, 0.0)
    scale = gamma * jax.lax.rsqrt(var + _EPS)
    shift = beta - mean * scale
    return scale, shift


def _stats_kernel(x_ref, o_ref):
    x = x_ref[...]
    s = jnp.sum(x, axis=0, keepdims=True)
    sq = jnp.sum(x * x, axis=0, keepdims=True)
    o_ref[...] = jnp.concatenate([s, sq], axis=0)[None]


def _conv_kernel(x_ref, stin_ref, gamma_ref, beta_ref, w_ref, y_ref, st_ref,
                 bl_ref, bm_ref, br_ref,
                 *, b, h, w, c_in, c_out, count, apply_silu):
    scale, shift = _fold_bn_from_partials(
        stin_ref[...], count, gamma_ref[...], beta_ref[...])

    s_acc = jnp.zeros((1, c_out), jnp.float32)
    sq_acc = jnp.zeros((1, c_out), jnp.float32)

    @pl.when(pl.program_id(0) == 0)
    def _():
        zrow2 = jnp.zeros((2, 1, w, c_in), jnp.bfloat16)
        for buf in (bl_ref, bm_ref, br_ref):
            buf[:, 0:1] = zrow2
            buf[:, h + 1:h + 2] = zrow2

    for k in range(b):
        p = k % 2
        a = x_ref[k].astype(jnp.float32).reshape(h * w, c_in)
        a = a * scale + shift
        if apply_silu:
            a = _silu(a)
        ab = a.astype(jnp.bfloat16)

        bm_ref[p, 1:h + 1] = ab.reshape(h, w, c_in)

        bl_ref[p, 1:h + 1] = pltpu.roll(ab, 1, 0).reshape(h, w, c_in)
        bl_ref[p, 1:h + 1, 0:1, :] = jnp.zeros((h, 1, c_in), jnp.bfloat16)

        br_ref[p, 1:h + 1] = pltpu.roll(ab, h * w - 1, 0).reshape(h, w, c_in)
        br_ref[p, 1:h + 1, w - 1:w, :] = jnp.zeros((h, 1, c_in), jnp.bfloat16)

        taps = [buf[p, kh:kh + h].reshape(h * w, c_in)
                for kh in range(3)
                for buf in (bl_ref, bm_ref, br_ref)]
        patches = jnp.concatenate(taps, axis=1)
        y = jnp.dot(patches, w_ref[...], preferred_element_type=jnp.float32)

        y_ref[k] = y.reshape(h, w, c_out).astype(y_ref.dtype)
        s_acc = s_acc + jnp.sum(y, axis=0, keepdims=True)
        sq_acc = sq_acc + jnp.sum(y * y, axis=0, keepdims=True)

    st_ref[...] = jnp.concatenate([s_acc, sq_acc], axis=0)[None]


def _bn_se_residual_kernel(y_ref, x_ref, stin_ref, gamma_ref, beta_ref,
                           wf1_ref, bf1_ref, wf2_ref, bf2_ref, o_ref,
                           *, b, h, w, c, count):
    scale, shift = _fold_bn_from_partials(
        stin_ref[...], count, gamma_ref[...], beta_ref[...])

    for k in range(b):
        z = y_ref[k].astype(jnp.float32).reshape(h * w, c)
        z = z * scale + shift

        pooled = jnp.sum(z, axis=0, keepdims=True) * (1.0 / (h * w))
        g = _silu(jnp.dot(pooled, wf1_ref[...],
                          preferred_element_type=jnp.float32) + bf1_ref[...])
        g = _sigmoid(jnp.dot(g, wf2_ref[...],
                             preferred_element_type=jnp.float32) + bf2_ref[...])

        x = x_ref[k].astype(jnp.float32).reshape(h * w, c)
        out = _silu(z * g + x)
        o_ref[k] = out.reshape(h, w, c).astype(o_ref.dtype)


def _channel_stats(x2d, c):
    rows = x2d.shape[0]
    n_chunks = 8
    while rows % n_chunks:
        n_chunks //= 2
    rt = rows // n_chunks
    return pl.pallas_call(
        _stats_kernel,
        grid=(n_chunks,),
        in_specs=[pl.BlockSpec((rt, c), lambda i: (i, 0))],
        out_specs=pl.BlockSpec((1, 2, c), lambda i: (i, 0, 0)),
        out_shape=jax.ShapeDtypeStruct((n_chunks, 2, c), jnp.float32),
        compiler_params=pltpu.CompilerParams(
            dimension_semantics=("parallel",),
            vmem_limit_bytes=_VMEM_LIMIT),
    )(x2d)


def _affine_conv3x3(x, stin, gamma, beta, wcol, *, count, apply_silu):
    n, h, w, c_in = x.shape
    c_out = wcol.shape[1]
    chunks = stin.shape[0]
    b = _IMGS_PER_STEP
    while n % b:
        b //= 2
    kfn = functools.partial(_conv_kernel, b=b, h=h, w=w, c_in=c_in,
                            c_out=c_out, count=count, apply_silu=apply_silu)
    return pl.pallas_call(
        kfn,
        grid=(n // b,),
        in_specs=[
            pl.BlockSpec((b, h, w, c_in), lambda i: (i, 0, 0, 0)),
            pl.BlockSpec((chunks, 2, c_in), lambda i: (0, 0, 0)),
            pl.BlockSpec((1, c_in), lambda i: (0, 0)),
            pl.BlockSpec((1, c_in), lambda i: (0, 0)),
            pl.BlockSpec((9 * c_in, c_out), lambda i: (0, 0)),
        ],
        out_specs=(
            pl.BlockSpec((b, h, w, c_out), lambda i: (i, 0, 0, 0)),
            pl.BlockSpec((1, 2, c_out), lambda i: (i, 0, 0)),
        ),
        out_shape=(
            jax.ShapeDtypeStruct((n, h, w, c_out), jnp.bfloat16),
            jax.ShapeDtypeStruct((n // b, 2, c_out), jnp.float32),
        ),
        scratch_shapes=[pltpu.VMEM((2, h + 2, w, c_in), jnp.bfloat16),
                        pltpu.VMEM((2, h + 2, w, c_in), jnp.bfloat16),
                        pltpu.VMEM((2, h + 2, w, c_in), jnp.bfloat16)],
        compiler_params=pltpu.CompilerParams(
            dimension_semantics=("arbitrary",),
            vmem_limit_bytes=_VMEM_LIMIT),
    )(x, stin, gamma, beta, wcol)


def _bn_se_residual(y, x, stin, gamma, beta, wf1, bf1, wf2, bf2, *, count):
    n, h, w, c = y.shape
    c_red = wf1.shape[1]
    chunks = stin.shape[0]
    b = _IMGS_PER_STEP
    while n % b:
        b //= 2
    kfn = functools.partial(_bn_se_residual_kernel, b=b, h=h, w=w, c=c,
                            count=count)
    return pl.pallas_call(
        kfn,
        grid=(n // b,),
        in_specs=[
            pl.BlockSpec((b, h, w, c), lambda i: (i, 0, 0, 0)),
            pl.BlockSpec((b, h, w, c), lambda i: (i, 0, 0, 0)),
            pl.BlockSpec((chunks, 2, c), lambda i: (0, 0, 0)),
            pl.BlockSpec((1, c), lambda i: (0, 0)),
            pl.BlockSpec((1, c), lambda i: (0, 0)),
            pl.BlockSpec((c, c_red), lambda i: (0, 0)),
            pl.BlockSpec((1, c_red), lambda i: (0, 0)),
            pl.BlockSpec((c_red, c), lambda i: (0, 0)),
            pl.BlockSpec((1, c), lambda i: (0, 0)),
        ],
        out_specs=pl.BlockSpec((b, h, w, c), lambda i: (i, 0, 0, 0)),
        out_shape=jax.ShapeDtypeStruct((n, h, w, c), jnp.float32),
        compiler_params=pltpu.CompilerParams(
            dimension_semantics=("parallel",),
            vmem_limit_bytes=_VMEM_LIMIT),
    )(y, x, stin, gamma, beta, wf1, bf1, wf2, bf2)


def kernel(x_nhwc, g0, b0, w1, g1, b1, w2, g2, b2, wf1, bf1, wf2, bf2):
    n, h, w, c = x_nhwc.shape
    count = float(n * h * w)

    w1col = w1.reshape(9 * c, c).astype(jnp.bfloat16)
    w2col = w2.reshape(9 * c, c).astype(jnp.bfloat16)

    st_x = _channel_stats(x_nhwc.reshape(n * h * w, c), c)
    y1, p1 = _affine_conv3x3(x_nhwc, st_x, g0, b0, w1col,
                             count=count, apply_silu=False)
    y2, p2 = _affine_conv3x3(y1, p1, g1, b1, w2col,
                             count=count, apply_silu=True)
    return _bn_se_residual(y2, x_nhwc, p2, g2, b2, wf1, bf1, wf2, bf2,
                           count=count)


# speedup vs baseline: 1.0080x; 1.0080x over previous
import functools

import jax
import jax.numpy as jnp
from jax.experimental import pallas as pl
from jax.experimental.pallas import tpu as pltpu

_EPS = 1e-5
_VMEM_LIMIT = 32 * 1024 * 1024
_IMGS_PER_STEP = 8


def _sigmoid(t):
    return 1.0 / (1.0 + jnp.exp(-t))


def _silu(t):
    return t * _sigmoid(t)


def _fold_bn_from_partials(st, count, gamma, beta):
    tot = jnp.sum(st, axis=0)
    mean = tot[0:1] / count
    var = jnp.maximum(tot[1:2] / count - mean * mean, 0.0)
    scale = gamma * jax.lax.rsqrt(var + _EPS)
    shift = beta - mean * scale
    return scale, shift


def _stats_kernel(x_ref, o_ref):
    x = x_ref[...]
    s = jnp.sum(x, axis=0, keepdims=True)
    sq = jnp.sum(x * x, axis=0, keepdims=True)
    o_ref[...] = jnp.concatenate([s, sq], axis=0)[None]


def _conv_kernel(x_ref, stin_ref, gamma_ref, beta_ref, w_ref, y_ref, st_ref,
                 bl_ref, bm_ref, br_ref,
                 *, b, h, w, c_in, c_out, count, apply_silu):
    scale, shift = _fold_bn_from_partials(
        stin_ref[...], count, gamma_ref[...], beta_ref[...])

    s_acc = jnp.zeros((1, c_out), jnp.float32)
    sq_acc = jnp.zeros((1, c_out), jnp.float32)

    @pl.when(pl.program_id(0) == 0)
    def _():
        zrow2 = jnp.zeros((2, 1, w, c_in), jnp.bfloat16)
        for buf in (bl_ref, bm_ref, br_ref):
            buf[:, 0:1] = zrow2
            buf[:, h + 1:h + 2] = zrow2

    for k in range(b):
        p = k % 2
        a = x_ref[k].astype(jnp.float32).reshape(h * w, c_in)
        a = a * scale + shift
        if apply_silu:
            a = _silu(a)
        ab = a.astype(jnp.bfloat16)

        bm_ref[p, 1:h + 1] = ab.reshape(h, w, c_in)

        bl_ref[p, 1:h + 1] = jnp.roll(ab, 1, axis=0).reshape(h, w, c_in)
        bl_ref[p, 1:h + 1, 0:1, :] = jnp.zeros((h, 1, c_in), jnp.bfloat16)

        br_ref[p, 1:h + 1] = jnp.roll(ab, -1, axis=0).reshape(h, w, c_in)
        br_ref[p, 1:h + 1, w - 1:w, :] = jnp.zeros((h, 1, c_in), jnp.bfloat16)

        taps = [buf[p, kh:kh + h].reshape(h * w, c_in)
                for kh in range(3)
                for buf in (bl_ref, bm_ref, br_ref)]
        patches = jnp.concatenate(taps, axis=1)
        y = jnp.dot(patches, w_ref[...], preferred_element_type=jnp.float32)

        y_ref[k] = y.reshape(h, w, c_out).astype(y_ref.dtype)
        s_acc = s_acc + jnp.sum(y, axis=0, keepdims=True)
        sq_acc = sq_acc + jnp.sum(y * y, axis=0, keepdims=True)

    st_ref[...] = jnp.concatenate([s_acc, sq_acc], axis=0)[None]


def _bn_se_residual_kernel(y_ref, x_ref, stin_ref, gamma_ref, beta_ref,
                           wf1_ref, bf1_ref, wf2_ref, bf2_ref, o_ref,
                           *, b, h, w, c, count):
    scale, shift = _fold_bn_from_partials(
        stin_ref[...], count, gamma_ref[...], beta_ref[...])

    for k in range(b):
        z = y_ref[k].astype(jnp.float32).reshape(h * w, c)
        z = z * scale + shift

        pooled = jnp.sum(z, axis=0, keepdims=True) * (1.0 / (h * w))
        g = _silu(jnp.dot(pooled, wf1_ref[...],
                          preferred_element_type=jnp.float32) + bf1_ref[...])
        g = _sigmoid(jnp.dot(g, wf2_ref[...],
                             preferred_element_type=jnp.float32) + bf2_ref[...])

        x = x_ref[k].astype(jnp.float32).reshape(h * w, c)
        out = _silu(z * g + x)
        o_ref[k] = out.reshape(h, w, c).astype(o_ref.dtype)


def _channel_stats(x2d, c):
    rows = x2d.shape[0]
    n_chunks = 8
    while rows % n_chunks:
        n_chunks //= 2
    rt = rows // n_chunks
    return pl.pallas_call(
        _stats_kernel,
        grid=(n_chunks,),
        in_specs=[pl.BlockSpec((rt, c), lambda i: (i, 0))],
        out_specs=pl.BlockSpec((1, 2, c), lambda i: (i, 0, 0)),
        out_shape=jax.ShapeDtypeStruct((n_chunks, 2, c), jnp.float32),
        compiler_params=pltpu.CompilerParams(
            dimension_semantics=("parallel",),
            vmem_limit_bytes=_VMEM_LIMIT),
    )(x2d)


def _affine_conv3x3(x, stin, gamma, beta, wcol, *, count, apply_silu):
    n, h, w, c_in = x.shape
    c_out = wcol.shape[1]
    chunks = stin.shape[0]
    b = _IMGS_PER_STEP
    while n % b:
        b //= 2
    kfn = functools.partial(_conv_kernel, b=b, h=h, w=w, c_in=c_in,
                            c_out=c_out, count=count, apply_silu=apply_silu)
    return pl.pallas_call(
        kfn,
        grid=(n // b,),
        in_specs=[
            pl.BlockSpec((b, h, w, c_in), lambda i: (i, 0, 0, 0)),
            pl.BlockSpec((chunks, 2, c_in), lambda i: (0, 0, 0)),
            pl.BlockSpec((1, c_in), lambda i: (0, 0)),
            pl.BlockSpec((1, c_in), lambda i: (0, 0)),
            pl.BlockSpec((9 * c_in, c_out), lambda i: (0, 0)),
        ],
        out_specs=(
            pl.BlockSpec((b, h, w, c_out), lambda i: (i, 0, 0, 0)),
            pl.BlockSpec((1, 2, c_out), lambda i: (i, 0, 0)),
        ),
        out_shape=(
            jax.ShapeDtypeStruct((n, h, w, c_out), jnp.bfloat16),
            jax.ShapeDtypeStruct((n // b, 2, c_out), jnp.float32),
        ),
        scratch_shapes=[pltpu.VMEM((2, h + 2, w, c_in), jnp.bfloat16),
                        pltpu.VMEM((2, h + 2, w, c_in), jnp.bfloat16),
                        pltpu.VMEM((2, h + 2, w, c_in), jnp.bfloat16)],
        compiler_params=pltpu.CompilerParams(
            dimension_semantics=("arbitrary",),
            vmem_limit_bytes=_VMEM_LIMIT),
    )(x, stin, gamma, beta, wcol)


def _bn_se_residual(y, x, stin, gamma, beta, wf1, bf1, wf2, bf2, *, count):
    n, h, w, c = y.shape
    c_red = wf1.shape[1]
    chunks = stin.shape[0]
    b = _IMGS_PER_STEP
    while n % b:
        b //= 2
    kfn = functools.partial(_bn_se_residual_kernel, b=b, h=h, w=w, c=c,
                            count=count)
    return pl.pallas_call(
        kfn,
        grid=(n // b,),
        in_specs=[
            pl.BlockSpec((b, h, w, c), lambda i: (i, 0, 0, 0)),
            pl.BlockSpec((b, h, w, c), lambda i: (i, 0, 0, 0)),
            pl.BlockSpec((chunks, 2, c), lambda i: (0, 0, 0)),
            pl.BlockSpec((1, c), lambda i: (0, 0)),
            pl.BlockSpec((1, c), lambda i: (0, 0)),
            pl.BlockSpec((c, c_red), lambda i: (0, 0)),
            pl.BlockSpec((1, c_red), lambda i: (0, 0)),
            pl.BlockSpec((c_red, c), lambda i: (0, 0)),
            pl.BlockSpec((1, c), lambda i: (0, 0)),
        ],
        out_specs=pl.BlockSpec((b, h, w, c), lambda i: (i, 0, 0, 0)),
        out_shape=jax.ShapeDtypeStruct((n, h, w, c), jnp.float32),
        compiler_params=pltpu.CompilerParams(
            dimension_semantics=("parallel",),
            vmem_limit_bytes=_VMEM_LIMIT),
    )(y, x, stin, gamma, beta, wf1, bf1, wf2, bf2)


def kernel(x_nhwc, g0, b0, w1, g1, b1, w2, g2, b2, wf1, bf1, wf2, bf2):
    n, h, w, c = x_nhwc.shape
    count = float(n * h * w)

    w1col = w1.reshape(9 * c, c).astype(jnp.bfloat16)
    w2col = w2.reshape(9 * c, c).astype(jnp.bfloat16)

    st_x = _channel_stats(x_nhwc.reshape(n * h * w, c), c)
    y1, p1 = _affine_conv3x3(x_nhwc, st_x, g0, b0, w1col,
                             count=count, apply_silu=False)
    y2, p2 = _affine_conv3x3(y1, p1, g1, b1, w2col,
                             count=count, apply_silu=True)
    return _bn_se_residual(y2, x_nhwc, p2, g2, b2, wf1, bf1, wf2, bf2,
                           count=count)
